# Initial kernel scaffold; baseline (speedup 1.0000x reference)
import jax
import jax.numpy as jnp
from jax.experimental import pallas as pl


def kernel(x, adj, W0, b0, gng0, gnb0, nng0, nnb0, W1, b1, gng1, gnb1, nng1, nnb1, W2, b2, gng2, gnb2, nng2, nnb2):
    raise NotImplementedError("write your pallas kernel here")


# speedup vs baseline: 1.0483x; 1.0483x over previous
import functools

import jax
import jax.numpy as jnp
from jax.experimental import pallas as pl

_HI = jax.lax.Precision.HIGHEST


def _transform_body(h_ref, w_ref, b_ref, o_ref):
    s = jnp.dot(h_ref[...], w_ref[...], preferred_element_type=jnp.float32,
                precision=_HI)
    o_ref[...] = (s + b_ref[...][None, :]).astype(o_ref.dtype)


def _transform(h, w, b, bi=1000):
    n, din = h.shape
    dout = w.shape[1]
    return pl.pallas_call(
        _transform_body,
        grid=(n // bi,),
        in_specs=[
            pl.BlockSpec((bi, din), lambda i: (i, 0)),
            pl.BlockSpec((din, dout), lambda i: (0, 0)),
            pl.BlockSpec((dout,), lambda i: (0,)),
        ],
        out_specs=pl.BlockSpec((bi, dout), lambda i: (i, 0)),
        out_shape=jax.ShapeDtypeStruct((n, dout), jnp.bfloat16),
    )(h, w, b)


def _spmm_body(adj_ref, s_ref, m_ref, sum_ref, sq_ref, *rest, cast):
    i = pl.program_id(0)
    a = adj_ref[...]
    if cast:
        a16 = a.astype(jnp.bfloat16)
        rest[0][...] = a16
    else:
        a16 = a
    prod = jnp.dot(a16, s_ref[...], preferred_element_type=jnp.float32)
    m_ref[...] = prod
    colsum = jnp.sum(prod, axis=0, keepdims=True)
    colsq = jnp.sum(prod * prod, axis=0, keepdims=True)

    @pl.when(i == 0)
    def _():
        sum_ref[...] = colsum
        sq_ref[...] = colsq

    @pl.when(i > 0)
    def _():
        sum_ref[...] += colsum
        sq_ref[...] += colsq


def _spmm(adj, s16, cast, bi):
    n, dout = s16.shape
    out_shapes = [
        jax.ShapeDtypeStruct((n, dout), jnp.float32),
        jax.ShapeDtypeStruct((1, dout), jnp.float32),
        jax.ShapeDtypeStruct((1, dout), jnp.float32),
    ]
    out_specs = [
        pl.BlockSpec((bi, dout), lambda i: (i, 0)),
        pl.BlockSpec((1, dout), lambda i: (0, 0)),
        pl.BlockSpec((1, dout), lambda i: (0, 0)),
    ]
    if cast:
        out_shapes.append(jax.ShapeDtypeStruct((n, n), jnp.bfloat16))
        out_specs.append(pl.BlockSpec((bi, n), lambda i: (i, 0)))
    res = pl.pallas_call(
        functools.partial(_spmm_body, cast=cast),
        grid=(n // bi,),
        in_specs=[
            pl.BlockSpec((bi, n), lambda i: (i, 0)),
            pl.BlockSpec((n, dout), lambda i: (0, 0)),
        ],
        out_specs=out_specs,
        out_shape=out_shapes,
    )(adj, s16)
    return res


def _norm_body(m_ref, sum_ref, sq_ref, gg_ref, gb_ref, ng_ref, nb_ref,
               *rest, n, last):
    mu = sum_ref[...] * (1.0 / n)
    var = sq_ref[...] * (1.0 / n) - mu * mu
    m = m_ref[...]
    g = (m - mu) * jax.lax.rsqrt(var + 1e-5) * gg_ref[...][None, :] \
        + gb_ref[...][None, :]
    nmu = jnp.mean(g, axis=1, keepdims=True)
    nvar = jnp.mean((g - nmu) ** 2, axis=1, keepdims=True)
    h = (g - nmu) * jax.lax.rsqrt(nvar + 1e-5) * ng_ref[...][None, :] \
        + nb_ref[...][None, :]
    if last:
        o_ref = rest[0]
        hmax = jnp.max(h, axis=1, keepdims=True)
        lse = jnp.log(jnp.sum(jnp.exp(h - hmax), axis=1, keepdims=True)) + hmax
        o_ref[...] = h - lse
    else:
        w_ref, b_ref, o_ref = rest
        s = jnp.dot(h, w_ref[...], preferred_element_type=jnp.float32,
                    precision=_HI)
        o_ref[...] = (s + b_ref[...][None, :]).astype(o_ref.dtype)


def _norm_transform(m, colsum, colsq, gg, gb, ng, nb, w=None, b=None,
                    bi=1000):
    n, dout = m.shape
    last = w is None
    in_specs = [
        pl.BlockSpec((bi, dout), lambda i: (i, 0)),
        pl.BlockSpec((1, dout), lambda i: (0, 0)),
        pl.BlockSpec((1, dout), lambda i: (0, 0)),
        pl.BlockSpec((dout,), lambda i: (0,)),
        pl.BlockSpec((dout,), lambda i: (0,)),
        pl.BlockSpec((dout,), lambda i: (0,)),
        pl.BlockSpec((dout,), lambda i: (0,)),
    ]
    args = [m, colsum, colsq, gg, gb, ng, nb]
    if last:
        out_shape = jax.ShapeDtypeStruct((n, dout), jnp.float32)
        out_spec = pl.BlockSpec((bi, dout), lambda i: (i, 0))
    else:
        dnext = w.shape[1]
        in_specs.append(pl.BlockSpec((dout, dnext), lambda i: (0, 0)))
        in_specs.append(pl.BlockSpec((dnext,), lambda i: (0,)))
        args.extend([w, b])
        out_shape = jax.ShapeDtypeStruct((n, dnext), jnp.bfloat16)
        out_spec = pl.BlockSpec((bi, dnext), lambda i: (i, 0))
    return pl.pallas_call(
        functools.partial(_norm_body, n=n, last=last),
        grid=(n // bi,),
        in_specs=in_specs,
        out_specs=out_spec,
        out_shape=out_shape,
    )(*args)


def kernel(x, adj, W0, b0, gng0, gnb0, nng0, nnb0,
           W1, b1, gng1, gnb1, nng1, nnb1,
           W2, b2, gng2, gnb2, nng2, nnb2):
    n = x.shape[0]
    bi0 = 200 if n % 200 == 0 else n
    bi = 400 if n % 400 == 0 else n

    s0 = _transform(x, W0, b0, bi=min(1000, n))
    m0, cs0, cq0, adj16 = _spmm(adj, s0, cast=True, bi=bi0)
    s1 = _norm_transform(m0, cs0, cq0, gng0, gnb0, nng0, nnb0, W1, b1,
                         bi=min(1000, n))
    m1, cs1, cq1 = _spmm(adj16, s1, cast=False, bi=bi)
    s2 = _norm_transform(m1, cs1, cq1, gng1, gnb1, nng1, nnb1, W2, b2,
                         bi=min(1000, n))
    m2, cs2, cq2 = _spmm(adj16, s2, cast=False, bi=bi)
    return _norm_transform(m2, cs2, cq2, gng2, gnb2, nng2, nnb2,
                           bi=min(1000, n))


# speedup vs baseline: 1.1528x; 1.0997x over previous
import functools

import jax
import jax.numpy as jnp
from jax.experimental import pallas as pl
from jax.experimental.pallas import tpu as pltpu

_EPS = 1e-5


def _graph_node_norm(m, csum, csq, gg, gb, ng, nb, n):
    mu = csum * (1.0 / n)
    var = csq * (1.0 / n) - mu * mu
    g = (m - mu) * jax.lax.rsqrt(var + _EPS) * gg[None, :] + gb[None, :]
    nmu = jnp.mean(g, axis=1, keepdims=True)
    nvar = jnp.mean((g - nmu) ** 2, axis=1, keepdims=True)
    return (g - nmu) * jax.lax.rsqrt(nvar + _EPS) * ng[None, :] + nb[None, :]


def _t0_body(x_ref, w0_ref, b0_ref, o_ref):
    s = jnp.dot(x_ref[...].astype(jnp.bfloat16),
                w0_ref[...].astype(jnp.bfloat16),
                preferred_element_type=jnp.float32)
    o_ref[...] = (s + b0_ref[...][None, :]).astype(jnp.bfloat16)


def _l0_body(s0_ref, adj_ref, gg_ref, gb_ref, ng_ref, nb_ref,
             w1_ref, b1_ref, adj16_ref, s1_ref, macc_ref,
             sum_ref, sq_ref, *, bi, n):
    i = pl.program_id(0)
    ni = pl.num_programs(0)

    a16 = adj_ref[...].astype(jnp.bfloat16)
    adj16_ref[...] = a16
    prod = jnp.dot(a16, s0_ref[...], preferred_element_type=jnp.float32)
    macc_ref[pl.ds(i * bi, bi), :] = prod
    colsum = jnp.sum(prod, axis=0, keepdims=True)
    colsq = jnp.sum(prod * prod, axis=0, keepdims=True)

    @pl.when(i == 0)
    def _():
        sum_ref[...] = colsum
        sq_ref[...] = colsq

    @pl.when(i > 0)
    def _():
        sum_ref[...] += colsum
        sq_ref[...] += colsq

    @pl.when(i == ni - 1)
    def _tail():
        h = _graph_node_norm(macc_ref[...], sum_ref[...], sq_ref[...],
                             gg_ref[...], gb_ref[...], ng_ref[...],
                             nb_ref[...], n)
        s = jnp.dot(h.astype(jnp.bfloat16), w1_ref[...].astype(jnp.bfloat16),
                    preferred_element_type=jnp.float32)
        s1_ref[...] = (s + b1_ref[...][None, :]).astype(jnp.bfloat16)


def _l12_body(adj16_ref, s1_ref, gg1_ref, gb1_ref, ng1_ref, nb1_ref,
              w2_ref, b2_ref, gg2_ref, gb2_ref, ng2_ref, nb2_ref,
              out_ref, s2_ref, macc_ref, sum1_ref, sq1_ref,
              sum2_ref, sq2_ref, *, bi, n, ni1, d2):
    t = pl.program_id(0)
    nt = pl.num_programs(0)

    @pl.when(t < ni1)
    def _layer1():
        prod = jnp.dot(adj16_ref[...], s1_ref[...],
                       preferred_element_type=jnp.float32)
        macc_ref[pl.ds(t * bi, bi), :] = prod
        colsum = jnp.sum(prod, axis=0, keepdims=True)
        colsq = jnp.sum(prod * prod, axis=0, keepdims=True)

        @pl.when(t == 0)
        def _():
            sum1_ref[...] = colsum
            sq1_ref[...] = colsq

        @pl.when(t > 0)
        def _():
            sum1_ref[...] += colsum
            sq1_ref[...] += colsq

    @pl.when(t == ni1)
    def _transition():
        h = _graph_node_norm(macc_ref[...], sum1_ref[...], sq1_ref[...],
                             gg1_ref[...], gb1_ref[...], ng1_ref[...],
                             nb1_ref[...], n)
        s = jnp.dot(h.astype(jnp.bfloat16), w2_ref[...].astype(jnp.bfloat16),
                    preferred_element_type=jnp.float32)
        s2_ref[...] = (s + b2_ref[...][None, :]).astype(jnp.bfloat16)

    @pl.when(t >= ni1)
    def _layer2():
        p = t - ni1
        prod = jnp.dot(adj16_ref[...], s2_ref[...],
                       preferred_element_type=jnp.float32)
        macc_ref[pl.ds(p * bi, bi), 0:d2] = prod
        colsum = jnp.sum(prod, axis=0, keepdims=True)
        colsq = jnp.sum(prod * prod, axis=0, keepdims=True)

        @pl.when(p == 0)
        def _():
            sum2_ref[...] = colsum
            sq2_ref[...] = colsq

        @pl.when(p > 0)
        def _():
            sum2_ref[...] += colsum
            sq2_ref[...] += colsq

    @pl.when(t == nt - 1)
    def _tail():
        h = _graph_node_norm(macc_ref[:, 0:d2], sum2_ref[...], sq2_ref[...],
                             gg2_ref[...], gb2_ref[...], ng2_ref[...],
                             nb2_ref[...], n)
        hmax = jnp.max(h, axis=1, keepdims=True)
        lse = jnp.log(jnp.sum(jnp.exp(h - hmax), axis=1, keepdims=True)) + hmax
        out_ref[...] = h - lse


def kernel(x, adj, W0, b0, gng0, gnb0, nng0, nnb0,
           W1, b1, gng1, gnb1, nng1, nnb1,
           W2, b2, gng2, gnb2, nng2, nnb2):
    n = x.shape[0]
    din = x.shape[1]
    d0 = W0.shape[1]
    d1 = W1.shape[1]
    d2 = W2.shape[1]
    bi0 = 200 if n % 200 == 0 else n
    bi = 400 if n % 400 == 0 else n
    ni0 = n // bi0
    ni = n // bi

    vec = lambda d: pl.BlockSpec((d,), lambda i: (0,))
    full = lambda r, c: pl.BlockSpec((r, c), lambda i: (0, 0))

    bt = 1000 if n % 1000 == 0 else n
    s0 = pl.pallas_call(
        _t0_body,
        grid=(n // bt,),
        in_specs=[
            pl.BlockSpec((bt, din), lambda i: (i, 0)),
            full(din, d0), vec(d0),
        ],
        out_specs=pl.BlockSpec((bt, d0), lambda i: (i, 0)),
        out_shape=jax.ShapeDtypeStruct((n, d0), jnp.bfloat16),
    )(x, W0, b0)

    adj16, s1 = pl.pallas_call(
        functools.partial(_l0_body, bi=bi0, n=n),
        grid=(ni0,),
        in_specs=[
            full(n, d0),
            pl.BlockSpec((bi0, n), lambda i: (i, 0)),
            vec(d0), vec(d0), vec(d0), vec(d0),
            full(d0, d1), vec(d1),
        ],
        out_specs=[
            pl.BlockSpec((bi0, n), lambda i: (i, 0)),
            full(n, d1),
        ],
        out_shape=[
            jax.ShapeDtypeStruct((n, n), jnp.bfloat16),
            jax.ShapeDtypeStruct((n, d1), jnp.bfloat16),
        ],
        scratch_shapes=[
            pltpu.VMEM((n, d0), jnp.float32),
            pltpu.VMEM((1, d0), jnp.float32),
            pltpu.VMEM((1, d0), jnp.float32),
        ],
    )(s0, adj, gng0, gnb0, nng0, nnb0, W1, b1)

    def adj_idx(t):
        return (jnp.where(t < ni, t, t - ni), 0)

    out = pl.pallas_call(
        functools.partial(_l12_body, bi=bi, n=n, ni1=ni, d2=d2),
        grid=(2 * ni,),
        in_specs=[
            pl.BlockSpec((bi, n), adj_idx),
            full(n, d1),
            vec(d1), vec(d1), vec(d1), vec(d1),
            full(d1, d2), vec(d2),
            vec(d2), vec(d2), vec(d2), vec(d2),
        ],
        out_specs=full(n, d2),
        out_shape=jax.ShapeDtypeStruct((n, d2), jnp.float32),
        scratch_shapes=[
            pltpu.VMEM((n, d2), jnp.bfloat16),
            pltpu.VMEM((n, d1), jnp.float32),
            pltpu.VMEM((1, d1), jnp.float32),
            pltpu.VMEM((1, d1), jnp.float32),
            pltpu.VMEM((1, d2), jnp.float32),
            pltpu.VMEM((1, d2), jnp.float32),
        ],
    )(adj16, s1, gng1, gnb1, nng1, nnb1, W2, b2, gng2, gnb2, nng2, nnb2)
    return out
